# baseline (speedup 1.0000x reference)
import functools

import jax
import jax.numpy as jnp
from jax import lax
from jax.experimental import pallas as pl
from jax.experimental.pallas import tpu as pltpu
from jax.experimental.pallas import tpu_sc as plsc

_CH = 128
_L = 16
_PK = 4


@functools.cache
def _build(B, E, D):
    info = plsc.get_sparse_core_info()
    nw = info.num_cores * info.num_subcores
    nc = info.num_cores
    bpw = B // nw
    n_chunks = bpw // _CH
    mesh = plsc.VectorSubcoreMesh(core_axis_name="c", subcore_axis_name="s")

    @functools.partial(
        pl.kernel,
        mesh=mesh,
        out_type=jax.ShapeDtypeStruct((B, D), jnp.float32),
        compiler_params=pltpu.CompilerParams(needs_layout_passes=False),
        scratch_types=[
            pltpu.VMEM((n_chunks, _CH), jnp.int32),
            pltpu.VMEM((n_chunks, _CH), jnp.int32),
            pltpu.VMEM((n_chunks, _CH), jnp.int32),
            pltpu.VMEM((n_chunks, _CH), jnp.int32),
            pltpu.VMEM((n_chunks, _CH), jnp.int32),
            pltpu.VMEM((n_chunks, _CH), jnp.int32),
            pltpu.VMEM((bpw,), jnp.float32),
            pltpu.VMEM((_CH, _PK * E), jnp.float32),
            pltpu.VMEM((_CH, _PK * E), jnp.float32),
            pltpu.VMEM((_CH, _PK * E), jnp.float32),
            pltpu.VMEM((_CH, D), jnp.float32),
            pltpu.SemaphoreType.DMA,
        ],
    )
    def k(a_gidx_hbm, p_gidx_hbm, s_gidx_hbm, a_off_hbm, p_off_hbm, s_off_hbm,
          dang_hbm, tab, out_hbm, a_gidx, p_gidx, s_gidx,
          a_off, p_off, s_off, d_v, a_rows, p_rows, s_rows, out_c, sem):
        wid = lax.axis_index("s") * nc + lax.axis_index("c")
        base = wid * bpw
        cbase = wid * n_chunks
        pltpu.sync_copy(a_gidx_hbm.at[pl.ds(cbase, n_chunks)], a_gidx)
        pltpu.sync_copy(p_gidx_hbm.at[pl.ds(cbase, n_chunks)], p_gidx)
        pltpu.sync_copy(s_gidx_hbm.at[pl.ds(cbase, n_chunks)], s_gidx)
        pltpu.sync_copy(a_off_hbm.at[pl.ds(cbase, n_chunks)], a_off)
        pltpu.sync_copy(p_off_hbm.at[pl.ds(cbase, n_chunks)], p_off)
        pltpu.sync_copy(s_off_hbm.at[pl.ds(cbase, n_chunks)], s_off)
        pltpu.sync_copy(dang_hbm.at[pl.ds(base, bpw)], d_v)
        lanes = lax.iota(jnp.int32, _L)
        dcol = jnp.full((_L,), 3 * E, jnp.int32)
        nh = E // _L
        for j in range(n_chunks):
            gathers = [
                pltpu.async_copy(tab.at[a_gidx.at[j]], a_rows, sem),
                pltpu.async_copy(tab.at[p_gidx.at[j]], p_rows, sem),
                pltpu.async_copy(tab.at[s_gidx.at[j]], s_rows, sem),
            ]
            for c in gathers:
                c.wait()

            @plsc.parallel_loop(0, _CH // _L, unroll=1)
            def _(blk):
                r0 = blk * _L
                for t, (buf, off) in enumerate(
                        ((a_rows, a_off), (p_rows, p_off), (s_rows, s_off))):
                    off16 = off[j, pl.ds(r0, _L)]
                    for rr in range(_L):
                        o = off16[rr]
                        r = r0 + rr
                        for h in range(nh):
                            out_c[r, pl.ds(t * E + h * _L, _L)] = (
                                buf[r, pl.ds(o + h * _L, _L)])
                d16 = d_v[pl.ds(j * _CH + r0, _L)]
                plsc.store_scatter(out_c, [r0 + lanes, dcol], d16)

            pltpu.sync_copy(out_c, out_hbm.at[pl.ds(base + j * _CH, _CH)])

    return k


def kernel(obs, action_embeddings, parent_embeddings, sibling_embeddings):
    B = obs.shape[0]
    N, E = action_embeddings.shape
    a = obs[:, 0].astype(jnp.int32)
    p = obs[:, 1].astype(jnp.int32)
    s = obs[:, 2].astype(jnp.int32)
    d = obs[:, 3]
    sh = (B // _CH, _CH)
    tab = jnp.concatenate(
        [jnp.concatenate([t[m::_PK] for m in range(_PK)], axis=1)
         for t in (action_embeddings, parent_embeddings, sibling_embeddings)],
        axis=0)
    gidx = [(x // _PK + t * (N // _PK)).reshape(sh)
            for t, x in enumerate((a, p, s))]
    offs = [((x % _PK) * E).reshape(sh) for x in (a, p, s)]
    return _build(B, E, 3 * E + 1)(
        gidx[0], gidx[1], gidx[2], offs[0], offs[1], offs[2], d, tab)


# speedup vs baseline: 8.8665x; 8.8665x over previous
import functools

import jax
import jax.numpy as jnp
from jax import lax
from jax.experimental import pallas as pl
from jax.experimental.pallas import tpu as pltpu
from jax.experimental.pallas import tpu_sc as plsc

_CH = 128
_L = 16
_W = 128


@functools.cache
def _build(B, E, D):
    info = plsc.get_sparse_core_info()
    nw = info.num_cores * info.num_subcores
    nc = info.num_cores
    bpw = B // nw
    n_chunks = bpw // _CH
    mesh = plsc.VectorSubcoreMesh(core_axis_name="c", subcore_axis_name="s")

    @functools.partial(
        pl.kernel,
        mesh=mesh,
        out_type=jax.ShapeDtypeStruct((B, D), jnp.float32),
        compiler_params=pltpu.CompilerParams(needs_layout_passes=False),
        scratch_types=[
            pltpu.VMEM((n_chunks, _CH), jnp.int32),
            pltpu.VMEM((n_chunks, _CH), jnp.int32),
            pltpu.VMEM((n_chunks, _CH), jnp.int32),
            pltpu.VMEM((bpw,), jnp.float32),
            pltpu.VMEM((_CH, _W), jnp.float32),
            pltpu.VMEM((_CH, _W), jnp.float32),
            pltpu.VMEM((_CH, _W), jnp.float32),
            pltpu.VMEM((_CH, D), jnp.float32),
            pltpu.SemaphoreType.DMA,
        ],
    )
    def k(a_idx_hbm, p_idx_hbm, s_idx_hbm, dang_hbm, a_tab, p_tab, s_tab,
          out_hbm, a_idx, p_idx, s_idx, d_v, a_rows, p_rows, s_rows, out_c,
          sem):
        wid = lax.axis_index("s") * nc + lax.axis_index("c")
        base = wid * bpw
        cbase = wid * n_chunks
        pltpu.sync_copy(a_idx_hbm.at[pl.ds(cbase, n_chunks)], a_idx)
        pltpu.sync_copy(p_idx_hbm.at[pl.ds(cbase, n_chunks)], p_idx)
        pltpu.sync_copy(s_idx_hbm.at[pl.ds(cbase, n_chunks)], s_idx)
        pltpu.sync_copy(dang_hbm.at[pl.ds(base, bpw)], d_v)
        lanes = lax.iota(jnp.int32, _L)
        dcol = jnp.full((_L,), 3 * E, jnp.int32)
        for j in range(n_chunks):
            gathers = [
                pltpu.async_copy(a_tab.at[a_idx.at[j]], a_rows, sem),
                pltpu.async_copy(p_tab.at[p_idx.at[j]], p_rows, sem),
                pltpu.async_copy(s_tab.at[s_idx.at[j]], s_rows, sem),
            ]
            for c in gathers:
                c.wait()

            @plsc.parallel_loop(0, _CH, unroll=4)
            def _(r):
                for t, buf in enumerate((a_rows, p_rows, s_rows)):
                    for h in range(E // _L):
                        out_c[r, pl.ds(t * E + h * _L, _L)] = (
                            buf[r, pl.ds(h * _L, _L)])

            for kk in range(_CH // _L):
                d16 = d_v[pl.ds(j * _CH + kk * _L, _L)]
                plsc.store_scatter(out_c, [lanes + kk * _L, dcol], d16)

            pltpu.sync_copy(out_c, out_hbm.at[pl.ds(base + j * _CH, _CH)])

    return k


def kernel(obs, action_embeddings, parent_embeddings, sibling_embeddings):
    B = obs.shape[0]
    N, E = action_embeddings.shape
    sh = (B // _CH, _CH)
    a = obs[:, 0].astype(jnp.int32).reshape(sh)
    p = obs[:, 1].astype(jnp.int32).reshape(sh)
    s = obs[:, 2].astype(jnp.int32).reshape(sh)
    d = obs[:, 3]
    pad = ((0, 0), (0, _W - E))
    tabs = [jnp.pad(t, pad) for t in
            (action_embeddings, parent_embeddings, sibling_embeddings)]
    return _build(B, E, 3 * E + 1)(a, p, s, d, tabs[0], tabs[1], tabs[2])


# speedup vs baseline: 9.7059x; 1.0947x over previous
import functools

import jax
import jax.numpy as jnp
from jax import lax
from jax.experimental import pallas as pl
from jax.experimental.pallas import tpu as pltpu
from jax.experimental.pallas import tpu_sc as plsc

_CH = 128
_L = 16
_W = 128


@functools.cache
def _build(B, E, D):
    info = plsc.get_sparse_core_info()
    nw = info.num_cores * info.num_subcores
    nc = info.num_cores
    bpw = B // nw
    n_chunks = bpw // _CH
    mesh = plsc.VectorSubcoreMesh(core_axis_name="c", subcore_axis_name="s")

    @functools.partial(
        pl.kernel,
        mesh=mesh,
        out_type=jax.ShapeDtypeStruct((B, D), jnp.float32),
        compiler_params=pltpu.CompilerParams(needs_layout_passes=False),
        scratch_types=[
            pltpu.VMEM((n_chunks, _CH), jnp.int32),
            pltpu.VMEM((n_chunks, _CH), jnp.int32),
            pltpu.VMEM((n_chunks, _CH), jnp.int32),
            pltpu.VMEM((bpw,), jnp.float32),
            pltpu.VMEM((_CH, _W), jnp.float32),
            pltpu.VMEM((_CH, _W), jnp.float32),
            pltpu.VMEM((_CH, _W), jnp.float32),
            pltpu.VMEM((_CH, D), jnp.float32),
            pltpu.SemaphoreType.DMA,
        ],
    )
    def k(a_idx_hbm, p_idx_hbm, s_idx_hbm, dang_hbm, tab,
          out_hbm, a_idx, p_idx, s_idx, d_v, a_rows, p_rows, s_rows, out_c,
          sem):
        wid = lax.axis_index("s") * nc + lax.axis_index("c")
        base = wid * bpw
        cbase = wid * n_chunks
        pltpu.sync_copy(a_idx_hbm.at[pl.ds(cbase, n_chunks)], a_idx)
        pltpu.sync_copy(p_idx_hbm.at[pl.ds(cbase, n_chunks)], p_idx)
        pltpu.sync_copy(s_idx_hbm.at[pl.ds(cbase, n_chunks)], s_idx)
        pltpu.sync_copy(dang_hbm.at[pl.ds(base, bpw)], d_v)
        lanes = lax.iota(jnp.int32, _L)
        dcol = jnp.full((_L,), 3 * E, jnp.int32)
        for j in range(n_chunks):
            gathers = [
                pltpu.async_copy(tab.at[a_idx.at[j]], a_rows, sem),
                pltpu.async_copy(tab.at[p_idx.at[j]], p_rows, sem),
                pltpu.async_copy(tab.at[s_idx.at[j]], s_rows, sem),
            ]
            for c in gathers:
                c.wait()

            @plsc.parallel_loop(0, _CH, unroll=4)
            def _(r):
                for t, buf in enumerate((a_rows, p_rows, s_rows)):
                    for h in range(E // _L):
                        c0 = t * E + h * _L
                        out_c[r, pl.ds(c0, _L)] = buf[r, pl.ds(c0, _L)]

            for kk in range(_CH // _L):
                d16 = d_v[pl.ds(j * _CH + kk * _L, _L)]
                plsc.store_scatter(out_c, [lanes + kk * _L, dcol], d16)

            pltpu.sync_copy(out_c, out_hbm.at[pl.ds(base + j * _CH, _CH)])

    return k


def kernel(obs, action_embeddings, parent_embeddings, sibling_embeddings):
    B = obs.shape[0]
    N, E = action_embeddings.shape
    sh = (B // _CH, _CH)
    a = obs[:, 0].astype(jnp.int32).reshape(sh)
    p = obs[:, 1].astype(jnp.int32).reshape(sh)
    s = obs[:, 2].astype(jnp.int32).reshape(sh)
    d = obs[:, 3]
    tab = jnp.concatenate(
        [action_embeddings, parent_embeddings, sibling_embeddings,
         jnp.zeros((N, _W - 3 * E), jnp.float32)], axis=1)
    return _build(B, E, 3 * E + 1)(a, p, s, d, tab)


# speedup vs baseline: 9.7210x; 1.0016x over previous
import functools

import jax
import jax.numpy as jnp
from jax import lax
from jax.experimental import pallas as pl
from jax.experimental.pallas import tpu as pltpu
from jax.experimental.pallas import tpu_sc as plsc

_CH = 128
_L = 16
_W = 128


@functools.cache
def _build(B, E, D):
    info = plsc.get_sparse_core_info()
    nw = info.num_cores * info.num_subcores
    nc = info.num_cores
    bpw = B // nw
    n_chunks = bpw // _CH
    mesh = plsc.VectorSubcoreMesh(core_axis_name="c", subcore_axis_name="s")

    @functools.partial(
        pl.kernel,
        mesh=mesh,
        out_type=jax.ShapeDtypeStruct((B, D), jnp.float32),
        compiler_params=pltpu.CompilerParams(needs_layout_passes=False),
        scratch_types=[
            pltpu.VMEM((n_chunks, _CH), jnp.int32),
            pltpu.VMEM((n_chunks, _CH), jnp.int32),
            pltpu.VMEM((n_chunks, _CH), jnp.int32),
            pltpu.VMEM((bpw,), jnp.float32),
            pltpu.VMEM((_CH, _W), jnp.float32),
            pltpu.VMEM((_CH, _W), jnp.float32),
            pltpu.VMEM((_CH, _W), jnp.float32),
            pltpu.VMEM((_CH, D), jnp.float32),
            pltpu.SemaphoreType.DMA,
        ],
    )
    def k(a_idx_hbm, p_idx_hbm, s_idx_hbm, dang_hbm, tab,
          out_hbm, a_idx, p_idx, s_idx, d_v, a_rows, p_rows, s_rows, out_c,
          sem):
        wid = lax.axis_index("s") * nc + lax.axis_index("c")
        base = wid * bpw
        cbase = wid * n_chunks
        pltpu.sync_copy(a_idx_hbm.at[pl.ds(cbase, n_chunks)], a_idx)
        pltpu.sync_copy(p_idx_hbm.at[pl.ds(cbase, n_chunks)], p_idx)
        pltpu.sync_copy(s_idx_hbm.at[pl.ds(cbase, n_chunks)], s_idx)
        pltpu.sync_copy(dang_hbm.at[pl.ds(base, bpw)], d_v)
        lanes = lax.iota(jnp.int32, _L)
        dcol = jnp.full((_L,), 3 * E, jnp.int32)
        for j in range(n_chunks):
            gathers = [
                pltpu.async_copy(tab.at[a_idx.at[j]], a_rows, sem),
                pltpu.async_copy(tab.at[p_idx.at[j]], p_rows, sem),
                pltpu.async_copy(tab.at[s_idx.at[j]], s_rows, sem),
            ]
            for c in gathers:
                c.wait()

            @plsc.parallel_loop(0, _CH, unroll=4)
            def _(r):
                for t, buf in enumerate((a_rows, p_rows, s_rows)):
                    for h in range(E // _L):
                        c0 = t * E + h * _L
                        out_c[r, pl.ds(c0, _L)] = buf[r, pl.ds(c0, _L)]

            for kk in range(_CH // _L):
                d16 = d_v[pl.ds(j * _CH + kk * _L, _L)]
                plsc.store_scatter(out_c, [lanes + kk * _L, dcol], d16)

            pltpu.sync_copy(out_c, out_hbm.at[pl.ds(base + j * _CH, _CH)])

    return k


def kernel(obs, action_embeddings, parent_embeddings, sibling_embeddings):
    B = obs.shape[0]
    N, E = action_embeddings.shape
    sh = (B // _CH, _CH)
    a = obs[:, 0].astype(jnp.int32).reshape(sh)
    p = obs[:, 1].astype(jnp.int32).reshape(sh)
    s = obs[:, 2].astype(jnp.int32).reshape(sh)
    d = obs[:, 3]
    tab = jnp.concatenate(
        [action_embeddings.T, parent_embeddings.T, sibling_embeddings.T,
         jnp.zeros((_W - 3 * E, N), jnp.float32)], axis=0).T
    return _build(B, E, 3 * E + 1)(a, p, s, d, tab)


# speedup vs baseline: 9.9083x; 1.0193x over previous
import functools

import jax
import jax.numpy as jnp
from jax import lax
from jax.experimental import pallas as pl
from jax.experimental.pallas import tpu as pltpu
from jax.experimental.pallas import tpu_sc as plsc

_CH = 128
_L = 16
_W = 128


@functools.cache
def _build(B, E, D):
    info = plsc.get_sparse_core_info()
    nw = info.num_cores * info.num_subcores
    nc = info.num_cores
    bpw = B // nw
    n_chunks = bpw // _CH
    mesh = plsc.VectorSubcoreMesh(core_axis_name="c", subcore_axis_name="s")

    @functools.partial(
        pl.kernel,
        mesh=mesh,
        out_type=jax.ShapeDtypeStruct((B, D), jnp.float32),
        compiler_params=pltpu.CompilerParams(needs_layout_passes=False),
        scratch_types=[
            pltpu.VMEM((n_chunks, _CH), jnp.int32),
            pltpu.VMEM((n_chunks, _CH), jnp.int32),
            pltpu.VMEM((n_chunks, _CH), jnp.int32),
            pltpu.VMEM((bpw,), jnp.float32),
            pltpu.VMEM((_CH, _W), jnp.float32),
            pltpu.VMEM((_CH, _W), jnp.float32),
            pltpu.VMEM((_CH, _W), jnp.float32),
            pltpu.VMEM((_CH, _W), jnp.float32),
            pltpu.VMEM((_CH, _W), jnp.float32),
            pltpu.VMEM((_CH, _W), jnp.float32),
            pltpu.VMEM((_CH, D), jnp.float32),
            pltpu.SemaphoreType.DMA,
            pltpu.SemaphoreType.DMA,
            pltpu.SemaphoreType.DMA,
        ],
    )
    def k(a_idx_hbm, p_idx_hbm, s_idx_hbm, dang_hbm, tab, out_hbm,
          a_idx, p_idx, s_idx, d_v, a0, p0, s0, a1, p1, s1, out_c,
          gsem0, gsem1, wsem):
        wid = lax.axis_index("s") * nc + lax.axis_index("c")
        base = wid * bpw
        cbase = wid * n_chunks
        pltpu.sync_copy(a_idx_hbm.at[pl.ds(cbase, n_chunks)], a_idx)
        pltpu.sync_copy(p_idx_hbm.at[pl.ds(cbase, n_chunks)], p_idx)
        pltpu.sync_copy(s_idx_hbm.at[pl.ds(cbase, n_chunks)], s_idx)
        pltpu.sync_copy(dang_hbm.at[pl.ds(base, bpw)], d_v)
        lanes = lax.iota(jnp.int32, _L)
        dcol = jnp.full((_L,), 3 * E, jnp.int32)
        bufs = ((a0, p0, s0), (a1, p1, s1))
        gsems = (gsem0, gsem1)

        def fire(j):
            bs, sem = bufs[j % 2], gsems[j % 2]
            return [
                pltpu.async_copy(tab.at[a_idx.at[j]], bs[0], sem),
                pltpu.async_copy(tab.at[p_idx.at[j]], bs[1], sem),
                pltpu.async_copy(tab.at[s_idx.at[j]], bs[2], sem),
            ]

        pending = fire(0)
        wh = None
        for j in range(n_chunks):
            nxt = fire(j + 1) if j + 1 < n_chunks else []
            for c in pending:
                c.wait()
            pending = nxt
            if wh is not None:
                wh.wait()

            cur = bufs[j % 2]

            @plsc.parallel_loop(0, _CH, unroll=4)
            def _(r):
                for t in range(3):
                    for h in range(E // _L):
                        c0 = t * E + h * _L
                        out_c[r, pl.ds(c0, _L)] = cur[t][r, pl.ds(c0, _L)]

            for kk in range(_CH // _L):
                d16 = d_v[pl.ds(j * _CH + kk * _L, _L)]
                plsc.store_scatter(out_c, [lanes + kk * _L, dcol], d16)

            wh = pltpu.async_copy(
                out_c, out_hbm.at[pl.ds(base + j * _CH, _CH)], wsem)
        wh.wait()

    return k


def kernel(obs, action_embeddings, parent_embeddings, sibling_embeddings):
    B = obs.shape[0]
    N, E = action_embeddings.shape
    sh = (B // _CH, _CH)
    a = obs[:, 0].astype(jnp.int32).reshape(sh)
    p = obs[:, 1].astype(jnp.int32).reshape(sh)
    s = obs[:, 2].astype(jnp.int32).reshape(sh)
    d = obs[:, 3]
    tab = jnp.concatenate(
        [action_embeddings, parent_embeddings, sibling_embeddings,
         jnp.zeros((N, _W - 3 * E), jnp.float32)], axis=1)
    return _build(B, E, 3 * E + 1)(a, p, s, d, tab)
